# baseline (speedup 1.0000x reference)
import functools

import jax
import jax.numpy as jnp
from jax import lax
from jax.experimental import pallas as pl
from jax.experimental.pallas import tpu as pltpu
from jax.experimental.pallas import tpu_sc as plsc

VOCAB = 1_000_000
EMBED_DIM = 64
BATCH = 16384
NUM_CORES = 2
NUM_SUBCORES = 16
LANES = 16
NUM_WORKERS = NUM_CORES * NUM_SUBCORES
ROWS_PER_WORKER = BATCH // NUM_WORKERS
GROUPS = ROWS_PER_WORKER // LANES


def _body(target_hbm, context_hbm, t_tab_hbm, c_tab_hbm, out_hbm,
          t_idx, c_idx, t_rows, c_rows, out_v, sem_t, sem_c):
  wid = lax.axis_index("s") * NUM_CORES + lax.axis_index("c")
  base = wid * ROWS_PER_WORKER

  pltpu.sync_copy(target_hbm.at[pl.ds(base, ROWS_PER_WORKER)], t_idx)
  pltpu.sync_copy(context_hbm.at[pl.ds(base, ROWS_PER_WORKER)], c_idx)

  gt = pltpu.async_copy(t_tab_hbm.at[t_idx], t_rows, sem_t)
  gc = pltpu.async_copy(c_tab_hbm.at[c_idx], c_rows, sem_c)
  gt.wait()
  gc.wait()

  lane = lax.iota(jnp.int32, LANES)

  def group(g, carry):
    row = g * LANES + lane
    acc = jnp.zeros((LANES,), jnp.float32)
    for j in range(EMBED_DIM):
      col = (lane + j) & (EMBED_DIM - 1)
      tv = plsc.load_gather(t_rows, [row, col])
      cv = plsc.load_gather(c_rows, [row, col])
      acc = acc + tv * cv
    out_v[pl.ds(g * LANES, LANES)] = acc
    return carry

  lax.fori_loop(0, GROUPS, group, 0)

  pltpu.sync_copy(out_v, out_hbm.at[pl.ds(base, ROWS_PER_WORKER)])


@jax.jit
def kernel(target, context, target_table, context_table):
  mesh = plsc.VectorSubcoreMesh(core_axis_name="c", subcore_axis_name="s")
  run = pl.kernel(
      _body,
      out_type=jax.ShapeDtypeStruct((BATCH,), jnp.float32),
      mesh=mesh,
      scratch_types=[
          pltpu.VMEM((ROWS_PER_WORKER,), jnp.int32),
          pltpu.VMEM((ROWS_PER_WORKER,), jnp.int32),
          pltpu.VMEM((ROWS_PER_WORKER, EMBED_DIM), jnp.float32),
          pltpu.VMEM((ROWS_PER_WORKER, EMBED_DIM), jnp.float32),
          pltpu.VMEM((ROWS_PER_WORKER,), jnp.float32),
          pltpu.SemaphoreType.DMA,
          pltpu.SemaphoreType.DMA,
      ],
      compiler_params=pltpu.CompilerParams(
          needs_layout_passes=False, use_tc_tiling_on_sc=False),
  )
  return run(target, context, target_table, context_table)
